# baseline (speedup 1.0000x reference)
import jax
import jax.numpy as jnp
from jax.experimental import pallas as pl
from jax.experimental.pallas import tpu as pltpu

_FACTOR = 20.0
_ROWS = 32768
_COLS = 512
_BLOCK_ROWS = 1024
_GRID = _ROWS // _BLOCK_ROWS


def _wmse_body(x_ref, t_ref, out_ref):
    i = pl.program_id(0)
    x = x_ref[...]
    t = t_ref[...]
    d = x - t
    sq = d * d
    fl = t > 0.0
    s_fl = jnp.sum(jnp.where(fl, sq, 0.0))
    s_un = jnp.sum(jnp.where(fl, 0.0, sq))
    c_fl = jnp.sum(jnp.where(fl, 1.0, 0.0))

    @pl.when(i == 0)
    def _init():
        out_ref[0] = 0.0
        out_ref[1] = 0.0
        out_ref[2] = 0.0

    out_ref[0] += s_fl
    out_ref[1] += s_un
    out_ref[2] += c_fl


def _finalize(sums, n):
    s_fl, s_un, c_fl = sums[0], sums[1], sums[2]
    c_un = n - c_fl
    flood_loss = jnp.where(c_fl > 0, s_fl / jnp.maximum(c_fl, 1.0), 0.0)
    unflood_loss = jnp.where(c_un > 0, s_un / jnp.maximum(c_un, 1.0), 0.0)
    loss = _FACTOR * flood_loss + unflood_loss
    return (loss, flood_loss, unflood_loss)


@jax.jit
def kernel(inputs, targets):
    n = inputs.size
    x = inputs.reshape(_ROWS, _COLS)
    t = targets.reshape(_ROWS, _COLS)
    sums = pl.pallas_call(
        _wmse_body,
        grid=(_GRID,),
        in_specs=[
            pl.BlockSpec((_BLOCK_ROWS, _COLS), lambda i: (i, 0)),
            pl.BlockSpec((_BLOCK_ROWS, _COLS), lambda i: (i, 0)),
        ],
        out_specs=pl.BlockSpec(memory_space=pltpu.SMEM),
        out_shape=jax.ShapeDtypeStruct((3,), jnp.float32),
    )(x, t)
    return _finalize(sums, jnp.float32(n))


# speedup vs baseline: 1.1836x; 1.1836x over previous
import jax
import jax.numpy as jnp
from jax.experimental import pallas as pl
from jax.experimental.pallas import tpu as pltpu

_FACTOR = 20.0
_ROWS = 32768
_COLS = 512
_BLOCK_ROWS = 4096
_GRID = _ROWS // _BLOCK_ROWS


def _wmse_body(x_ref, t_ref, out_ref):
    i = pl.program_id(0)
    x = x_ref[...]
    t = t_ref[...]
    d = x - t
    sq = d * d
    fl = t > 0.0
    s_fl = jnp.sum(jnp.where(fl, sq, 0.0))
    s_un = jnp.sum(jnp.where(fl, 0.0, sq))
    c_fl = jnp.sum(jnp.where(fl, 1.0, 0.0))

    @pl.when(i == 0)
    def _init():
        out_ref[0] = 0.0
        out_ref[1] = 0.0
        out_ref[2] = 0.0

    out_ref[0] += s_fl
    out_ref[1] += s_un
    out_ref[2] += c_fl


def _finalize(sums, n):
    s_fl, s_un, c_fl = sums[0], sums[1], sums[2]
    c_un = n - c_fl
    flood_loss = jnp.where(c_fl > 0, s_fl / jnp.maximum(c_fl, 1.0), 0.0)
    unflood_loss = jnp.where(c_un > 0, s_un / jnp.maximum(c_un, 1.0), 0.0)
    loss = _FACTOR * flood_loss + unflood_loss
    return (loss, flood_loss, unflood_loss)


@jax.jit
def kernel(inputs, targets):
    n = inputs.size
    x = inputs.reshape(_ROWS, _COLS)
    t = targets.reshape(_ROWS, _COLS)
    sums = pl.pallas_call(
        _wmse_body,
        grid=(_GRID,),
        in_specs=[
            pl.BlockSpec((_BLOCK_ROWS, _COLS), lambda i: (i, 0)),
            pl.BlockSpec((_BLOCK_ROWS, _COLS), lambda i: (i, 0)),
        ],
        out_specs=pl.BlockSpec(memory_space=pltpu.SMEM),
        out_shape=jax.ShapeDtypeStruct((3,), jnp.float32),
    )(x, t)
    return _finalize(sums, jnp.float32(n))
